# baseline (speedup 1.0000x reference)
import functools

import jax
import jax.numpy as jnp
from jax.experimental import pallas as pl

MARGIN = 0.5
EPS = 1e-05


def _loss_block(a_ref, tcol_ref, b_ref, trow_ref, out_ref):
    i = pl.program_id(0)
    a = a_ref[...]
    b = b_ref[...]
    sim = jax.lax.dot_general(
        a, b, (((1,), (1,)), ((), ())), preferred_element_type=jnp.float32
    )
    trow = trow_ref[...]
    m = sim.shape[1]
    acc = jnp.zeros((8, m), jnp.float32)
    for k in range(sim.shape[0] // 8):
        s = sim[k * 8:(k + 1) * 8, :]
        same = tcol_ref[k * 8:(k + 1) * 8, :] == trow
        t = jnp.where(same, 1.0 - s, s)
        thr = jnp.where(same, jnp.float32(EPS), jnp.float32(MARGIN))
        acc = jnp.where(t > thr, acc + t, acc)
    partial = jnp.sum(acc, keepdims=True)

    @pl.when(i == 0)
    def _init():
        out_ref[...] = jnp.zeros_like(out_ref)

    out_ref[...] += partial


@functools.partial(jax.jit, static_argnames=("block_n",))
def _contrastive_loss(inputs_col, targets_col, inputs_row, target_row, block_n=2048):
    n, d = inputs_col.shape
    m = inputs_row.shape[0]
    tcol = targets_col.reshape(n, 1)
    trow = target_row.reshape(1, m)
    total = pl.pallas_call(
        _loss_block,
        grid=(n // block_n,),
        in_specs=[
            pl.BlockSpec((block_n, d), lambda i: (i, 0)),
            pl.BlockSpec((block_n, 1), lambda i: (i, 0)),
            pl.BlockSpec((m, d), lambda i: (0, 0)),
            pl.BlockSpec((1, m), lambda i: (0, 0)),
        ],
        out_specs=pl.BlockSpec((1, 1), lambda i: (0, 0)),
        out_shape=jax.ShapeDtypeStruct((1, 1), jnp.float32),
    )(inputs_col, tcol, inputs_row, trow)
    return total[0, 0] / n


def kernel(inputs_col, targets_col, inputs_row, target_row):
    return _contrastive_loss(inputs_col, targets_col, inputs_row, target_row)


# speedup vs baseline: 1.0582x; 1.0582x over previous
import functools

import jax
import jax.numpy as jnp
from jax.experimental import pallas as pl

MARGIN = 0.5
EPS = 1e-05


def _loss_block(a_ref, tcol_ref, b_ref, trow_ref, out_ref):
    i = pl.program_id(0)
    a = a_ref[...]
    b = b_ref[...]
    sim = jax.lax.dot_general(
        a, b, (((1,), (1,)), ((), ())), preferred_element_type=jnp.float32
    )
    trow = trow_ref[...]
    m = sim.shape[1]
    acc = jnp.zeros((8, m), jnp.float32)
    for k in range(sim.shape[0] // 8):
        s = sim[k * 8:(k + 1) * 8, :]
        same = tcol_ref[k * 8:(k + 1) * 8, :] == trow
        t = jnp.where(same, 1.0 - s, s)
        thr = jnp.where(same, jnp.float32(EPS), jnp.float32(MARGIN))
        acc = jnp.where(t > thr, acc + t, acc)
    partial = jnp.sum(acc, keepdims=True)

    @pl.when(i == 0)
    def _init():
        out_ref[...] = jnp.zeros_like(out_ref)

    out_ref[...] += partial


@functools.partial(jax.jit, static_argnames=("block_m",))
def _contrastive_loss(inputs_col, targets_col, inputs_row, target_row, block_m=1024):
    n, d = inputs_col.shape
    m = inputs_row.shape[0]
    tcol = targets_col.reshape(n, 1)
    trow = target_row.reshape(1, m)
    total = pl.pallas_call(
        _loss_block,
        grid=(m // block_m,),
        in_specs=[
            pl.BlockSpec((n, d), lambda i: (0, 0)),
            pl.BlockSpec((n, 1), lambda i: (0, 0)),
            pl.BlockSpec((block_m, d), lambda i: (i, 0)),
            pl.BlockSpec((1, block_m), lambda i: (0, i)),
        ],
        out_specs=pl.BlockSpec((1, 1), lambda i: (0, 0)),
        out_shape=jax.ShapeDtypeStruct((1, 1), jnp.float32),
    )(inputs_col, tcol, inputs_row, trow)
    return total[0, 0] / n


def kernel(inputs_col, targets_col, inputs_row, target_row):
    return _contrastive_loss(inputs_col, targets_col, inputs_row, target_row)
